# Initial kernel scaffold; baseline (speedup 1.0000x reference)
import jax
import jax.numpy as jnp
from jax.experimental import pallas as pl


def kernel(inputp, W):
    raise NotImplementedError("write your pallas kernel here")


# speedup vs baseline: 14.0400x; 14.0400x over previous
import functools

import jax
import jax.numpy as jnp
from jax.experimental import pallas as pl
from jax.experimental.pallas import tpu as pltpu

_B = 4096
_N = 128
_K = 64
_ROWS = 8
_ITERS = 12


def _spike_sum(z, iters):
    hi = jnp.max(z, axis=1)
    lo = jnp.zeros_like(hi)
    kf = jnp.float32(_K)
    for _ in range(iters):
        mid = 0.5 * (lo + hi)
        cnt = jnp.sum((z <= mid[:, None, :]).astype(jnp.float32), axis=1)
        ge = cnt >= kf
        hi = jnp.where(ge, mid, hi)
        lo = jnp.where(ge, lo, mid)
    t = hi[:, None, :]
    lt = z < t
    cnt_lt = jnp.sum(lt.astype(jnp.float32), axis=1)
    s_lt = jnp.sum(jnp.where(lt, z, 0.0), axis=1)
    return (s_lt + (kf - cnt_lt) * hi) * (1.0 / _K)


def _body(x_ref, w_ref, o_ref):
    x = x_ref[...]
    w = w_ref[...]
    a = jnp.maximum(3.0 + x, 0.0)
    b = jnp.maximum(3.0 - x, 0.0)
    p = jnp.maximum(3.0 + w, 0.0)
    m = jnp.maximum(3.0 - w, 0.0)
    u = jnp.concatenate([a, b], axis=1)[:, :, None]
    v_p = jnp.concatenate([p, m], axis=0)[None, :, :]
    v_m = jnp.concatenate([m, p], axis=0)[None, :, :]
    s_plus = _spike_sum(u + v_p, _ITERS)
    s_minus = _spike_sum(u + v_m, _ITERS)
    o_ref[...] = s_plus - s_minus


@jax.jit
def kernel(inputp, W):
    grid = _B // _ROWS
    return pl.pallas_call(
        _body,
        grid=(grid,),
        in_specs=[
            pl.BlockSpec((_ROWS, _N), lambda i: (i, 0)),
            pl.BlockSpec((_N, _N), lambda i: (0, 0)),
        ],
        out_specs=pl.BlockSpec((_ROWS, _N), lambda i: (i, 0)),
        out_shape=jax.ShapeDtypeStruct((_B, _N), jnp.float32),
    )(inputp, W)
